# baseline (speedup 1.0000x reference)
import functools

import jax
import jax.numpy as jnp
from jax.experimental import pallas as pl
from jax.experimental.pallas import tpu as pltpu


def _gap_kernel(x_ref, o_ref, *, spatial, inv_hw, out_rows):
    bn = x_ref.shape[2]
    c = x_ref.shape[3]
    x3 = x_ref[...].reshape(spatial, bn, c)
    s = jnp.sum(x3, axis=0, dtype=jnp.float32) * inv_hw
    o_ref[...] = s.reshape(out_rows, 128).astype(o_ref.dtype)


def kernel(x):
    n, c, h, w = x.shape
    spatial = h * w
    inv_hw = 1.0 / float(spatial)
    itemsize = jnp.dtype(x.dtype).itemsize

    xt = jnp.transpose(x, (2, 3, 0, 1))

    bn = n
    while bn > 8 and spatial * bn * c * itemsize > (4 << 20):
        bn //= 2
    grid = n // bn

    out = pl.pallas_call(
        functools.partial(
            _gap_kernel,
            spatial=spatial,
            inv_hw=inv_hw,
            out_rows=bn * c // 128,
        ),
        out_shape=jax.ShapeDtypeStruct((n * c // 128, 128), x.dtype),
        grid=(grid,),
        in_specs=[pl.BlockSpec((h, w, bn, c), lambda i: (0, 0, i, 0))],
        out_specs=pl.BlockSpec((bn * c // 128, 128), lambda i: (i, 0)),
        compiler_params=pltpu.CompilerParams(
            dimension_semantics=("parallel",),
            vmem_limit_bytes=64 << 20,
        ),
        cost_estimate=pl.CostEstimate(
            flops=n * c * spatial,
            transcendentals=0,
            bytes_accessed=(n * c * spatial + n * c) * itemsize,
        ),
    )(xt)

    return out.reshape(n, c, 1, 1)


# speedup vs baseline: 1.1741x; 1.1741x over previous
import functools

import jax
import jax.numpy as jnp
from jax.experimental import pallas as pl
from jax.experimental.pallas import tpu as pltpu


def _gap_kernel(x_ref, o_ref, *, spatial, inv_hw, out_rows):
    bn = x_ref.shape[2]
    c = x_ref.shape[3]
    x3 = x_ref[...].reshape(spatial, bn, c)
    s = jnp.sum(x3, axis=0, dtype=jnp.float32) * inv_hw
    o_ref[...] = s.reshape(out_rows, 128).astype(o_ref.dtype)


def kernel(x):
    n, c, h, w = x.shape
    spatial = h * w
    inv_hw = 1.0 / float(spatial)
    itemsize = jnp.dtype(x.dtype).itemsize

    xt = jnp.transpose(x, (2, 3, 0, 1))

    bn = n
    while bn > 8 and spatial * bn * c * itemsize > (8 << 20):
        bn //= 2
    grid = n // bn

    out = pl.pallas_call(
        functools.partial(
            _gap_kernel,
            spatial=spatial,
            inv_hw=inv_hw,
            out_rows=bn * c // 128,
        ),
        out_shape=jax.ShapeDtypeStruct((n * c // 128, 128), x.dtype),
        grid=(grid,),
        in_specs=[pl.BlockSpec((h, w, bn, c), lambda i: (0, 0, i, 0))],
        out_specs=pl.BlockSpec((bn * c // 128, 128), lambda i: (i, 0)),
        compiler_params=pltpu.CompilerParams(
            dimension_semantics=("parallel",),
            vmem_limit_bytes=64 << 20,
        ),
        cost_estimate=pl.CostEstimate(
            flops=n * c * spatial,
            transcendentals=0,
            bytes_accessed=(n * c * spatial + n * c) * itemsize,
        ),
    )(xt)

    return out.reshape(n, c, 1, 1)
